# baseline (speedup 1.0000x reference)
import functools

import jax
import jax.numpy as jnp
from jax import lax
from jax.experimental import pallas as pl
from jax.experimental.pallas import tpu as pltpu
from jax.experimental.pallas import tpu_sc as plsc

D = 256
CHUNK = 200


@functools.lru_cache(maxsize=None)
def _make_gather2(B: int):
    info = plsc.get_sparse_core_info()
    nc, ns = info.num_cores, info.num_subcores
    nw = nc * ns
    nchunk = B // CHUNK
    assert nchunk * CHUNK == B
    units_per_worker = -(-nchunk // nw)
    mesh = plsc.VectorSubcoreMesh(core_axis_name="c", subcore_axis_name="s")

    @functools.partial(
        pl.kernel,
        mesh=mesh,
        out_type=[
            jax.ShapeDtypeStruct((B, D), jnp.float32),
            jax.ShapeDtypeStruct((B, D), jnp.float32),
        ],
        scratch_types=[
            pltpu.VMEM((CHUNK,), jnp.int32),
            pltpu.VMEM((CHUNK,), jnp.int32),
            pltpu.VMEM((CHUNK, D), jnp.float32),
            pltpu.VMEM((CHUNK, D), jnp.float32),
            pltpu.SemaphoreType.DMA,
            pltpu.SemaphoreType.DMA,
        ],
    )
    def gather2(uids_hbm, iids_hbm, eg_hbm, ed_hbm, outu_hbm, outi_hbm,
                uidx_v, iidx_v, urow_v, irow_v, usem, isem):
        wid = lax.axis_index("s") * nc + lax.axis_index("c")

        def body(k, carry):
            g = wid * units_per_worker + k

            @pl.when(g < nchunk)
            def _():
                base = g * CHUNK
                pltpu.sync_copy(uids_hbm.at[pl.ds(base, CHUNK)], uidx_v)
                pltpu.sync_copy(iids_hbm.at[pl.ds(base, CHUNK)], iidx_v)
                ucp = pltpu.async_copy(eg_hbm.at[uidx_v], urow_v, usem)
                icp = pltpu.async_copy(ed_hbm.at[iidx_v], irow_v, isem)
                ucp.wait()
                pltpu.sync_copy(urow_v, outu_hbm.at[pl.ds(base, CHUNK)])
                icp.wait()
                pltpu.sync_copy(irow_v, outi_hbm.at[pl.ds(base, CHUNK)])

            return carry

        lax.fori_loop(0, units_per_worker, body, 0)

    return gather2


def _mlp_body(u_ref, i_ref, w1a_ref, w1b_ref, b1_ref, w2_ref, b2_ref,
              w3t_ref, b3_ref, out_ref):
    bf = jnp.bfloat16
    h = (
        jnp.dot(u_ref[...].astype(bf), w1a_ref[...].astype(bf),
                preferred_element_type=jnp.float32)
        + jnp.dot(i_ref[...].astype(bf), w1b_ref[...].astype(bf),
                  preferred_element_type=jnp.float32)
        + b1_ref[...]
    )
    h = jnp.maximum(h, 0.0)
    h = jnp.dot(h.astype(bf), w2_ref[...].astype(bf),
                preferred_element_type=jnp.float32) + b2_ref[...]
    h = jnp.maximum(h, 0.0)
    out_ref[...] = (
        lax.dot_general(w3t_ref[...].astype(bf), h.astype(bf),
                        (((1,), (1,)), ((), ())),
                        preferred_element_type=jnp.float32)
        + b3_ref[...]
    )[None]


@functools.lru_cache(maxsize=None)
def _make_mlp(B: int, R: int, interpret: bool = False):
    nb = B // R
    assert nb * R == B
    rep = lambda i: (0, 0)
    return pl.pallas_call(
        _mlp_body,
        grid=(nb,),
        in_specs=[
            pl.BlockSpec((R, D), lambda i: (i, 0)),
            pl.BlockSpec((R, D), lambda i: (i, 0)),
            pl.BlockSpec((D, D), rep),
            pl.BlockSpec((D, D), rep),
            pl.BlockSpec((1, D), rep),
            pl.BlockSpec((D, D), rep),
            pl.BlockSpec((1, D), rep),
            pl.BlockSpec((1, D), rep),
            pl.BlockSpec((1, 1), rep),
        ],
        out_specs=pl.BlockSpec((1, 1, R), lambda i: (i, 0, 0)),
        out_shape=jax.ShapeDtypeStruct((nb, 1, R), jnp.float32),
        interpret=interpret,
    )


def kernel(uids, iids, E_g, E_d, W1, b1, W2, b2, W3, b3):
    B = uids.shape[0]
    u_emb, i_emb = _make_gather2(B)(
        uids.astype(jnp.int32), iids.astype(jnp.int32), E_g, E_d)
    out = _make_mlp(B, 1000)(
        u_emb, i_emb,
        W1[:D], W1[D:],
        b1.reshape(1, D),
        W2, b2.reshape(1, D),
        W3.reshape(1, D),
        b3.reshape(1, 1),
    )
    return (out.reshape(1, B), E_g, E_d)


# speedup vs baseline: 1.1224x; 1.1224x over previous
import functools

import jax
import jax.numpy as jnp
from jax import lax
from jax.experimental import pallas as pl
from jax.experimental.pallas import tpu as pltpu
from jax.experimental.pallas import tpu_sc as plsc

D = 256
CHUNK = 200


@functools.lru_cache(maxsize=None)
def _make_gather2(B: int):
    info = plsc.get_sparse_core_info()
    nc, ns = info.num_cores, info.num_subcores
    nw = nc * ns
    nchunk = B // CHUNK
    assert nchunk * CHUNK == B
    units_per_worker = -(-nchunk // nw)
    mesh = plsc.VectorSubcoreMesh(core_axis_name="c", subcore_axis_name="s")

    @functools.partial(
        pl.kernel,
        mesh=mesh,
        out_type=[
            jax.ShapeDtypeStruct((B, D), jnp.float32),
            jax.ShapeDtypeStruct((B, D), jnp.float32),
        ],
        scratch_types=[
            pltpu.VMEM((CHUNK,), jnp.int32),
            pltpu.VMEM((CHUNK,), jnp.int32),
            pltpu.VMEM((CHUNK, D), jnp.float32),
            pltpu.VMEM((CHUNK, D), jnp.float32),
            pltpu.SemaphoreType.DMA,
            pltpu.SemaphoreType.DMA,
        ],
    )
    def gather2(uids_hbm, iids_hbm, eg_hbm, ed_hbm, outu_hbm, outi_hbm,
                uidx_v, iidx_v, urow_v, irow_v, usem, isem):
        wid = lax.axis_index("s") * nc + lax.axis_index("c")

        def body(k, carry):
            g = wid * units_per_worker + k

            @pl.when(g < nchunk)
            def _():
                base = g * CHUNK
                pltpu.sync_copy(uids_hbm.at[pl.ds(base, CHUNK)], uidx_v)
                pltpu.sync_copy(iids_hbm.at[pl.ds(base, CHUNK)], iidx_v)
                ucp = pltpu.async_copy(eg_hbm.at[uidx_v], urow_v, usem)
                icp = pltpu.async_copy(ed_hbm.at[iidx_v], irow_v, isem)
                ucp.wait()
                pltpu.sync_copy(urow_v, outu_hbm.at[pl.ds(base, CHUNK)])
                icp.wait()
                pltpu.sync_copy(irow_v, outi_hbm.at[pl.ds(base, CHUNK)])

            return carry

        lax.fori_loop(0, units_per_worker, body, 0)

    return gather2


def _mlp_body(u_ref, i_ref, w1a_ref, w1b_ref, b1_ref, w2_ref, b2_ref,
              w3t_ref, b3_ref, out_ref):
    bf = jnp.bfloat16
    h = (
        jnp.dot(u_ref[...].astype(bf), w1a_ref[...].astype(bf),
                preferred_element_type=jnp.float32)
        + jnp.dot(i_ref[...].astype(bf), w1b_ref[...].astype(bf),
                  preferred_element_type=jnp.float32)
        + b1_ref[...]
    )
    h = jnp.maximum(h, 0.0)
    h = jnp.dot(h.astype(bf), w2_ref[...].astype(bf),
                preferred_element_type=jnp.float32) + b2_ref[...]
    h = jnp.maximum(h, 0.0)
    out_ref[...] = (
        lax.dot_general(w3t_ref[...].astype(bf), h.astype(bf),
                        (((1,), (1,)), ((), ())),
                        preferred_element_type=jnp.float32)
        + b3_ref[...]
    )[None]


@functools.lru_cache(maxsize=None)
def _make_mlp(B: int, R: int, interpret: bool = False):
    nb = B // R
    assert nb * R == B
    rep = lambda i: (0, 0)
    return pl.pallas_call(
        _mlp_body,
        grid=(nb,),
        in_specs=[
            pl.BlockSpec((R, D), lambda i: (i, 0)),
            pl.BlockSpec((R, D), lambda i: (i, 0)),
            pl.BlockSpec((D, D), rep),
            pl.BlockSpec((D, D), rep),
            pl.BlockSpec((1, D), rep),
            pl.BlockSpec((D, D), rep),
            pl.BlockSpec((1, D), rep),
            pl.BlockSpec((1, D), rep),
            pl.BlockSpec((1, 1), rep),
        ],
        out_specs=pl.BlockSpec((1, 1, R), lambda i: (i, 0, 0)),
        out_shape=jax.ShapeDtypeStruct((nb, 1, R), jnp.float32),
        interpret=interpret,
    )


def kernel(uids, iids, E_g, E_d, W1, b1, W2, b2, W3, b3):
    B = uids.shape[0]
    u_emb, i_emb = _make_gather2(B)(
        uids.astype(jnp.int32), iids.astype(jnp.int32), E_g, E_d)
    out = _make_mlp(B, 4000)(
        u_emb, i_emb,
        W1[:D], W1[D:],
        b1.reshape(1, D),
        W2, b2.reshape(1, D),
        W3.reshape(1, D),
        b3.reshape(1, 1),
    )
    return (out.reshape(1, B), E_g, E_d)


# speedup vs baseline: 1.1247x; 1.0020x over previous
import functools

import jax
import jax.numpy as jnp
from jax import lax
from jax.experimental import pallas as pl
from jax.experimental.pallas import tpu as pltpu
from jax.experimental.pallas import tpu_sc as plsc

D = 256
CHUNK = 200


@functools.lru_cache(maxsize=None)
def _make_gather2(B: int):
    info = plsc.get_sparse_core_info()
    nc, ns = info.num_cores, info.num_subcores
    nw = nc * ns
    nchunk = B // CHUNK
    assert nchunk * CHUNK == B
    units_per_worker = -(-nchunk // nw)
    mesh = plsc.VectorSubcoreMesh(core_axis_name="c", subcore_axis_name="s")

    @functools.partial(
        pl.kernel,
        mesh=mesh,
        out_type=[
            jax.ShapeDtypeStruct((B, D), jnp.float32),
            jax.ShapeDtypeStruct((B, D), jnp.float32),
        ],
        scratch_types=[
            pltpu.VMEM((CHUNK,), jnp.int32),
            pltpu.VMEM((CHUNK,), jnp.int32),
            pltpu.VMEM((CHUNK, D), jnp.float32),
            pltpu.VMEM((CHUNK, D), jnp.float32),
            pltpu.SemaphoreType.DMA,
            pltpu.SemaphoreType.DMA,
        ],
    )
    def gather2(uids_hbm, iids_hbm, eg_hbm, ed_hbm, outu_hbm, outi_hbm,
                uidx_v, iidx_v, urow_v, irow_v, usem, isem):
        wid = lax.axis_index("s") * nc + lax.axis_index("c")

        def body(k, carry):
            g = wid * units_per_worker + k

            @pl.when(g < nchunk)
            def _():
                base = g * CHUNK
                pltpu.sync_copy(uids_hbm.at[pl.ds(base, CHUNK)], uidx_v)
                pltpu.sync_copy(iids_hbm.at[pl.ds(base, CHUNK)], iidx_v)
                ucp = pltpu.async_copy(eg_hbm.at[uidx_v], urow_v, usem)
                icp = pltpu.async_copy(ed_hbm.at[iidx_v], irow_v, isem)
                ucp.wait()
                pltpu.sync_copy(urow_v, outu_hbm.at[pl.ds(base, CHUNK)])
                icp.wait()
                pltpu.sync_copy(irow_v, outi_hbm.at[pl.ds(base, CHUNK)])

            return carry

        lax.fori_loop(0, units_per_worker, body, 0)

    return gather2


def _mlp_body(u_ref, i_ref, w1a_ref, w1b_ref, b1_ref, w2_ref, b2_ref,
              w3t_ref, b3_ref, out_ref):
    bf = jnp.bfloat16
    h = (
        jnp.dot(u_ref[...].astype(bf), w1a_ref[...].astype(bf),
                preferred_element_type=jnp.float32)
        + jnp.dot(i_ref[...].astype(bf), w1b_ref[...].astype(bf),
                  preferred_element_type=jnp.float32)
        + b1_ref[...]
    )
    h = jnp.maximum(h, 0.0)
    h = jnp.dot(h.astype(bf), w2_ref[...].astype(bf),
                preferred_element_type=jnp.float32) + b2_ref[...]
    h = jnp.maximum(h, 0.0)
    out_ref[...] = (
        lax.dot_general(w3t_ref[...].astype(bf), h.astype(bf),
                        (((1,), (1,)), ((), ())),
                        preferred_element_type=jnp.float32)
        + b3_ref[...]
    )[None]


@functools.lru_cache(maxsize=None)
def _make_mlp(B: int, R: int, interpret: bool = False):
    nb = B // R
    assert nb * R == B
    rep = lambda i: (0, 0)
    return pl.pallas_call(
        _mlp_body,
        grid=(nb,),
        in_specs=[
            pl.BlockSpec((R, D), lambda i: (i, 0)),
            pl.BlockSpec((R, D), lambda i: (i, 0)),
            pl.BlockSpec((D, D), rep),
            pl.BlockSpec((D, D), rep),
            pl.BlockSpec((1, D), rep),
            pl.BlockSpec((D, D), rep),
            pl.BlockSpec((1, D), rep),
            pl.BlockSpec((1, D), rep),
            pl.BlockSpec((1, 1), rep),
        ],
        out_specs=pl.BlockSpec((1, 1, R), lambda i: (i, 0, 0)),
        out_shape=jax.ShapeDtypeStruct((nb, 1, R), jnp.float32),
        interpret=interpret,
    )


def kernel(uids, iids, E_g, E_d, W1, b1, W2, b2, W3, b3):
    B = uids.shape[0]
    u_emb, i_emb = _make_gather2(B)(
        uids.astype(jnp.int32), iids.astype(jnp.int32), E_g, E_d)
    out = _make_mlp(B, 5000)(
        u_emb, i_emb,
        W1[:D], W1[D:],
        b1.reshape(1, D),
        W2, b2.reshape(1, D),
        W3.reshape(1, D),
        b3.reshape(1, 1),
    )
    return (out.reshape(1, B), E_g, E_d)
